# baseline (speedup 1.0000x reference)
import functools

import jax
import jax.numpy as jnp
from jax import lax
from jax.experimental import pallas as pl
from jax.experimental.pallas import tpu as pltpu
from jax.experimental.pallas import tpu_sc as plsc

_NC = 2
_NS = 16
_CH = 128
_NH = 5000
_PADR = 1144
_ACC = _NH + _PADR
_NBUF = 2

_F32 = jnp.float32


def _mesh():
    return plsc.VectorSubcoreMesh(core_axis_name="c", subcore_axis_name="s")


def _fill_vmem_f32(buf, rows, val):
    @pl.loop(0, rows)
    def _(i):
        @pl.loop(0, 8)
        def _(j):
            buf[i, pl.ds(j * 16, 16)] = jnp.full((16,), val, _F32)


def _zero_acc(acc, zbuf, sid, rpt):
    base = sid * rpt

    @pl.loop(0, rpt // 64)
    def _(r):
        pltpu.sync_copy(zbuf, acc.at[pl.ds(base + r * 64, 64)])


def _acc_to_out(acc, out_hbm, cid, sid, rpt):
    base = sid * rpt

    @pl.loop(0, rpt // 64)
    def _(r):
        pltpu.sync_copy(acc.at[pl.ds(base + r * 64, 64)],
                        out_hbm.at[cid, pl.ds(base + r * 64, 64)])


def _sc_deg(dst_rs):
    nch = dst_rs.shape[2]
    rpt = _ACC // _NS

    @functools.partial(
        pl.kernel,
        out_type=jax.ShapeDtypeStruct((_NC, _ACC, 128), _F32),
        mesh=_mesh(),
        scratch_types=[
            pltpu.VMEM((nch, _CH), jnp.int32),
            pltpu.VMEM((_CH, 128), _F32),
            pltpu.VMEM((64, 128), _F32),
            pltpu.VMEM_SHARED((_ACC, 128), _F32),
        ],
    )
    def k(dst_hbm, out_hbm, didx, ones, zbuf, acc):
        cid = lax.axis_index("c")
        sid = lax.axis_index("s")

        _fill_vmem_f32(ones, _CH, 1.0)
        _fill_vmem_f32(zbuf, 64, 0.0)
        _zero_acc(acc, zbuf, sid, rpt)
        pltpu.sync_copy(dst_hbm.at[cid, sid], didx)
        plsc.subcore_barrier()

        @pl.loop(0, nch)
        def _(c):
            pltpu.sync_copy(ones, acc.at[didx.at[c]], add=True)

        plsc.subcore_barrier()
        _acc_to_out(acc, out_hbm, cid, sid, rpt)

    return k(dst_rs)


def _sc_agg(y, src_rs, dst_rs):
    nch = src_rs.shape[1]
    rpt = _ACC // _NS

    @functools.partial(
        pl.kernel,
        out_type=jax.ShapeDtypeStruct((_NC, _ACC, 128), _F32),
        mesh=_mesh(),
        scratch_types=[
            pltpu.VMEM((nch, _CH), jnp.int32),
            pltpu.VMEM((nch, _CH), jnp.int32),
            pltpu.VMEM((64, 128), _F32),
            pltpu.VMEM_SHARED((_ACC, 128), _F32),
        ] + [pltpu.VMEM((_CH, 128), _F32)] * _NBUF
          + [pltpu.SemaphoreType.DMA] * _NBUF,
    )
    def k(y_hbm, src_hbm, dst_hbm, out_hbm, sidx, didx, zbuf, acc, *bufsem):
        bufs = bufsem[:_NBUF]
        sg = bufsem[_NBUF:2 * _NBUF]
        cid = lax.axis_index("c")
        sid = lax.axis_index("s")

        _fill_vmem_f32(zbuf, 64, 0.0)
        _zero_acc(acc, zbuf, sid, rpt)
        pltpu.sync_copy(src_hbm.at[sid], sidx)
        pltpu.sync_copy(dst_hbm.at[cid, sid], didx)
        plsc.subcore_barrier()

        for j in range(_NBUF):
            pltpu.async_copy(y_hbm.at[sidx.at[j]], bufs[j], sg[j])

        @pl.loop(0, nch // _NBUF)
        def _(i):
            c0 = _NBUF * i
            for j in range(_NBUF):
                c = c0 + j
                pltpu.make_async_copy(y_hbm.at[sidx.at[c]],
                                      bufs[j], sg[j]).wait()
                pltpu.sync_copy(bufs[j], acc.at[didx.at[c]], add=True)

                @pl.when(c + _NBUF < nch)
                def _():
                    pltpu.async_copy(y_hbm.at[sidx.at[c + _NBUF]],
                                     bufs[j], sg[j])

        plsc.subcore_barrier()
        _acc_to_out(acc, out_hbm, cid, sid, rpt)

    return k(y, src_rs, dst_rs)


def _dinv_block(degp_ref):
    d = degp_ref[0, :, 0:1] + 1.0
    return lax.rsqrt(d)


def _node_map3(i):
    return (i // 5, i % 5, 0)


def _tc_p1(x, W1, degp, bn):
    n = x.shape[0]

    def body(x_ref, w_ref, degp_ref, y_ref):
        dinv = _dinv_block(degp_ref)
        y_ref[...] = dinv * jnp.dot(x_ref[...], w_ref[...],
                                    preferred_element_type=_F32)

    return pl.pallas_call(
        body,
        grid=(n // bn,),
        in_specs=[
            pl.BlockSpec((bn, 128), lambda i: (i, 0)),
            pl.BlockSpec((128, 128), lambda i: (0, 0)),
            pl.BlockSpec((1, bn, 128), _node_map3),
        ],
        out_specs=pl.BlockSpec((bn, 128), lambda i: (i, 0)),
        out_shape=jax.ShapeDtypeStruct((n, 128), _F32),
    )(x, W1, degp)


def _tc_p3(agg1, y1, degp, W2, b1, bn):
    n = y1.shape[0]

    def body(agg_ref, y_ref, degp_ref, w_ref, b_ref, y2_ref):
        dinv = _dinv_block(degp_ref)
        s = agg_ref[0] + y_ref[...]
        z = jnp.maximum(dinv * s + b_ref[...], 0.0)
        y2_ref[...] = dinv * jnp.dot(z, w_ref[...], preferred_element_type=_F32)

    return pl.pallas_call(
        body,
        grid=(n // bn,),
        in_specs=[
            pl.BlockSpec((1, bn, 128), _node_map3),
            pl.BlockSpec((bn, 128), lambda i: (i, 0)),
            pl.BlockSpec((1, bn, 128), _node_map3),
            pl.BlockSpec((128, 128), lambda i: (0, 0)),
            pl.BlockSpec((1, 128), lambda i: (0, 0)),
        ],
        out_specs=pl.BlockSpec((bn, 128), lambda i: (i, 0)),
        out_shape=jax.ShapeDtypeStruct((n, 128), _F32),
    )(agg1, y1, degp, W2, b1)


def _tc_p5(agg2, y2, degp, b2, bn):
    n = y2.shape[0]

    def body(agg_ref, y_ref, degp_ref, b_ref, o_ref):
        dinv = _dinv_block(degp_ref)
        s = agg_ref[0] + y_ref[...]
        o_ref[...] = dinv * s + b_ref[...]

    return pl.pallas_call(
        body,
        grid=(n // bn,),
        in_specs=[
            pl.BlockSpec((1, bn, 128), _node_map3),
            pl.BlockSpec((bn, 128), lambda i: (i, 0)),
            pl.BlockSpec((1, bn, 128), _node_map3),
            pl.BlockSpec((1, 128), lambda i: (0, 0)),
        ],
        out_specs=pl.BlockSpec((bn, 128), lambda i: (i, 0)),
        out_shape=jax.ShapeDtypeStruct((n, 128), _F32),
    )(agg2, y2, degp, b2)


def kernel(x, edge_index, W1, b1, W2, b2):
    n = x.shape[0]
    e = edge_index.shape[1]
    src = edge_index[0]
    dst = edge_index[1]
    assert n == _NC * _NH and n % 1000 == 0

    nch = -(-e // (_NS * _CH))
    nch = -(-nch // _NBUF) * _NBUF
    npad = _NS * nch * _CH - e

    pad_src = (jnp.arange(npad, dtype=jnp.int32) * 37) % n
    pad_dst = jnp.full((npad,), n, jnp.int32)
    src_full = jnp.concatenate([src, pad_src])
    dst_full = jnp.concatenate([dst, pad_dst])
    src_rs = src_full.reshape(_NS, nch, _CH)

    spread = _NH + (jnp.arange(e + npad, dtype=jnp.int32) % _PADR)
    halves = []
    for c in range(_NC):
        local = dst_full - c * _NH
        ok = (local >= 0) & (local < _NH)
        halves.append(jnp.where(ok, local, spread).reshape(_NS, nch, _CH))
    dst_rs = jnp.stack(halves, axis=0)

    bn = 1000
    b1r = b1.reshape(1, 128)
    b2r = b2.reshape(1, 128)

    degp = _sc_deg(dst_rs)
    y1 = _tc_p1(x, W1, degp, bn)
    agg1 = _sc_agg(y1, src_rs, dst_rs)
    y2 = _tc_p3(agg1, y1, degp, W2, b1r, bn)
    agg2 = _sc_agg(y2, src_rs, dst_rs)
    out = _tc_p5(agg2, y2, degp, b2r, bn)
    return out


# speedup vs baseline: 1.0049x; 1.0049x over previous
import functools

import jax
import jax.numpy as jnp
from jax import lax
from jax.experimental import pallas as pl
from jax.experimental.pallas import tpu as pltpu
from jax.experimental.pallas import tpu_sc as plsc

_NC = 2
_NS = 16
_CH = 128
_NH = 5000
_PADR = 120
_ACC = _NH + _PADR
_NBUF = 2

_F32 = jnp.float32


def _mesh():
    return plsc.VectorSubcoreMesh(core_axis_name="c", subcore_axis_name="s")


def _fill_vmem_f32(buf, rows, val):
    @pl.loop(0, rows)
    def _(i):
        @pl.loop(0, 8)
        def _(j):
            buf[i, pl.ds(j * 16, 16)] = jnp.full((16,), val, _F32)


def _zero_acc(acc, zbuf, sid, rpt):
    base = sid * rpt

    @pl.loop(0, rpt // 64)
    def _(r):
        pltpu.sync_copy(zbuf, acc.at[pl.ds(base + r * 64, 64)])


_OUTR = 5120


def _acc_to_out(acc, out_hbm, cid, sid):
    rpt = _OUTR // _NS
    base = sid * rpt

    @pl.loop(0, rpt // 64)
    def _(r):
        pltpu.sync_copy(acc.at[pl.ds(base + r * 64, 64)],
                        out_hbm.at[cid, pl.ds(base + r * 64, 64)])


def _sc_deg(dst_rs):
    nch = dst_rs.shape[2]
    rpt = _ACC // _NS

    @functools.partial(
        pl.kernel,
        out_type=pltpu.HBM((_NC, _OUTR, 128), _F32),
        mesh=_mesh(),
        scratch_types=[
            pltpu.VMEM((nch, _CH), jnp.int32),
            pltpu.VMEM((_CH, 128), _F32),
            pltpu.VMEM((64, 128), _F32),
            pltpu.VMEM_SHARED((_ACC, 128), _F32),
        ],
    )
    def k(dst_hbm, out_hbm, didx, ones, zbuf, acc):
        cid = lax.axis_index("c")
        sid = lax.axis_index("s")

        _fill_vmem_f32(ones, _CH, 1.0)
        _fill_vmem_f32(zbuf, 64, 0.0)
        _zero_acc(acc, zbuf, sid, rpt)
        pltpu.sync_copy(dst_hbm.at[cid, sid], didx)
        plsc.subcore_barrier()

        @pl.loop(0, nch)
        def _(c):
            pltpu.sync_copy(ones, acc.at[didx.at[c]], add=True)

        plsc.subcore_barrier()
        _acc_to_out(acc, out_hbm, cid, sid)

    return k(dst_rs)


def _sc_agg(y, src_rs, dst_rs):
    nch = src_rs.shape[1]
    rpt = _ACC // _NS

    @functools.partial(
        pl.kernel,
        out_type=pltpu.HBM((_NC, _OUTR, 128), _F32),
        mesh=_mesh(),
        scratch_types=[
            pltpu.VMEM((nch, _CH), jnp.int32),
            pltpu.VMEM((nch, _CH), jnp.int32),
            pltpu.VMEM((64, 128), _F32),
            pltpu.VMEM_SHARED((_ACC, 128), _F32),
        ] + [pltpu.VMEM((_CH, 128), _F32)] * _NBUF
          + [pltpu.SemaphoreType.DMA] * _NBUF,
    )
    def k(y_hbm, src_hbm, dst_hbm, out_hbm, sidx, didx, zbuf, acc, *bufsem):
        bufs = bufsem[:_NBUF]
        sg = bufsem[_NBUF:2 * _NBUF]
        cid = lax.axis_index("c")
        sid = lax.axis_index("s")

        _fill_vmem_f32(zbuf, 64, 0.0)
        _zero_acc(acc, zbuf, sid, rpt)
        pltpu.sync_copy(src_hbm.at[sid], sidx)
        pltpu.sync_copy(dst_hbm.at[cid, sid], didx)
        plsc.subcore_barrier()

        for j in range(_NBUF):
            pltpu.async_copy(y_hbm.at[sidx.at[j]], bufs[j], sg[j])

        @pl.loop(0, nch // _NBUF)
        def _(i):
            c0 = _NBUF * i
            for j in range(_NBUF):
                c = c0 + j
                pltpu.make_async_copy(y_hbm.at[sidx.at[c]],
                                      bufs[j], sg[j]).wait()
                pltpu.sync_copy(bufs[j], acc.at[didx.at[c]], add=True)

                @pl.when(c + _NBUF < nch)
                def _():
                    pltpu.async_copy(y_hbm.at[sidx.at[c + _NBUF]],
                                     bufs[j], sg[j])

        plsc.subcore_barrier()
        _acc_to_out(acc, out_hbm, cid, sid)

    return k(y, src_rs, dst_rs)


def _dinv_block(degp_ref):
    d = degp_ref[0, :, 0:1] + 1.0
    return lax.rsqrt(d)


def _node_map3(i):
    return (i // 5, i % 5, 0)


def _tc_p1(x, W1, degp, bn):
    n = x.shape[0]

    def body(x_ref, w_ref, degp_ref, y_ref):
        dinv = _dinv_block(degp_ref)
        y_ref[...] = dinv * jnp.dot(x_ref[...], w_ref[...],
                                    preferred_element_type=_F32)

    return pl.pallas_call(
        body,
        grid=(n // bn,),
        in_specs=[
            pl.BlockSpec((bn, 128), lambda i: (i, 0)),
            pl.BlockSpec((128, 128), lambda i: (0, 0)),
            pl.BlockSpec((1, bn, 128), _node_map3),
        ],
        out_specs=pl.BlockSpec((bn, 128), lambda i: (i, 0)),
        out_shape=jax.ShapeDtypeStruct((n, 128), _F32),
    )(x, W1, degp)


def _tc_p3(agg1, y1, degp, W2, b1, bn):
    n = y1.shape[0]

    def body(agg_ref, y_ref, degp_ref, w_ref, b_ref, y2_ref):
        dinv = _dinv_block(degp_ref)
        s = agg_ref[0] + y_ref[...]
        z = jnp.maximum(dinv * s + b_ref[...], 0.0)
        y2_ref[...] = dinv * jnp.dot(z, w_ref[...], preferred_element_type=_F32)

    return pl.pallas_call(
        body,
        grid=(n // bn,),
        in_specs=[
            pl.BlockSpec((1, bn, 128), _node_map3),
            pl.BlockSpec((bn, 128), lambda i: (i, 0)),
            pl.BlockSpec((1, bn, 128), _node_map3),
            pl.BlockSpec((128, 128), lambda i: (0, 0)),
            pl.BlockSpec((1, 128), lambda i: (0, 0)),
        ],
        out_specs=pl.BlockSpec((bn, 128), lambda i: (i, 0)),
        out_shape=jax.ShapeDtypeStruct((n, 128), _F32),
    )(agg1, y1, degp, W2, b1)


def _tc_p5(agg2, y2, degp, b2, bn):
    n = y2.shape[0]

    def body(agg_ref, y_ref, degp_ref, b_ref, o_ref):
        dinv = _dinv_block(degp_ref)
        s = agg_ref[0] + y_ref[...]
        o_ref[...] = dinv * s + b_ref[...]

    return pl.pallas_call(
        body,
        grid=(n // bn,),
        in_specs=[
            pl.BlockSpec((1, bn, 128), _node_map3),
            pl.BlockSpec((bn, 128), lambda i: (i, 0)),
            pl.BlockSpec((1, bn, 128), _node_map3),
            pl.BlockSpec((1, 128), lambda i: (0, 0)),
        ],
        out_specs=pl.BlockSpec((bn, 128), lambda i: (i, 0)),
        out_shape=jax.ShapeDtypeStruct((n, 128), _F32),
    )(agg2, y2, degp, b2)


def kernel(x, edge_index, W1, b1, W2, b2):
    n = x.shape[0]
    e = edge_index.shape[1]
    src = edge_index[0]
    dst = edge_index[1]
    assert n == _NC * _NH and n % 1000 == 0

    nch = -(-e // (_NS * _CH))
    nch = -(-nch // _NBUF) * _NBUF
    npad = _NS * nch * _CH - e

    pad_src = (jnp.arange(npad, dtype=jnp.int32) * 37) % n
    pad_dst = jnp.full((npad,), n, jnp.int32)
    src_full = jnp.concatenate([src, pad_src])
    dst_full = jnp.concatenate([dst, pad_dst])
    src_rs = src_full.reshape(_NS, nch, _CH)

    spread = _NH + (jnp.arange(e + npad, dtype=jnp.int32) % _PADR)
    halves = []
    for c in range(_NC):
        local = dst_full - c * _NH
        ok = (local >= 0) & (local < _NH)
        halves.append(jnp.where(ok, local, spread).reshape(_NS, nch, _CH))
    dst_rs = jnp.stack(halves, axis=0)

    bn = 1000
    b1r = b1.reshape(1, 128)
    b2r = b2.reshape(1, 128)

    degp = _sc_deg(dst_rs)
    y1 = _tc_p1(x, W1, degp, bn)
    agg1 = _sc_agg(y1, src_rs, dst_rs)
    y2 = _tc_p3(agg1, y1, degp, W2, b1r, bn)
    agg2 = _sc_agg(y2, src_rs, dst_rs)
    out = _tc_p5(agg2, y2, degp, b2r, bn)
    return out


# speedup vs baseline: 1.5215x; 1.5140x over previous
import functools

import jax
import jax.numpy as jnp
from jax import lax
from jax.experimental import pallas as pl
from jax.experimental.pallas import tpu as pltpu
from jax.experimental.pallas import tpu_sc as plsc

_NC = 2
_NS = 16
_CH = 128
_PADR = 240
_NBUF = 2

_F32 = jnp.float32


def _mesh():
    return plsc.VectorSubcoreMesh(core_axis_name="c", subcore_axis_name="s")


def _fill_vmem_f32(buf, rows, val):
    @pl.loop(0, rows)
    def _(i):
        @pl.loop(0, 8)
        def _(j):
            buf[i, pl.ds(j * 16, 16)] = jnp.full((16,), val, _F32)


def _zero_acc(acc, zbuf, sid, rpt):
    base = sid * rpt

    @pl.loop(0, rpt // 32)
    def _(r):
        pltpu.sync_copy(zbuf, acc.at[pl.ds(base + r * 32, 32)])


def _acc_to_out(acc, out_hbm, cid, sid, rpt):
    base = sid * rpt

    @pl.loop(0, rpt // 64)
    def _(r):
        pltpu.sync_copy(acc.at[pl.ds(base + r * 64, 64)],
                        out_hbm.at[cid, pl.ds(base + r * 64, 64)])


def _sc_deg(dst_rs, out_ref, accn):
    nch = dst_rs.shape[1]
    nch_h = nch // _NC
    rpt = accn // _NS

    @functools.partial(
        pl.kernel,
        out_type=(),
        mesh=_mesh(),
        scratch_types=[
            pltpu.VMEM((nch_h // 2, _CH), jnp.int32),
            pltpu.VMEM((_CH, 128), _F32),
            pltpu.VMEM((32, 128), _F32),
            pltpu.VMEM_SHARED((accn, 128), _F32),
        ],
    )
    def k(dst_hbm, out_hbm, didx, ones, zbuf, acc):
        cid = lax.axis_index("c")
        sid = lax.axis_index("s")
        nchq = nch_h // 2

        _fill_vmem_f32(ones, _CH, 1.0)
        _fill_vmem_f32(zbuf, 32, 0.0)
        _zero_acc(acc, zbuf, sid, rpt)
        plsc.subcore_barrier()

        for h in range(2):
            off = cid * nch_h + h * nchq
            pltpu.sync_copy(dst_hbm.at[sid, pl.ds(off, nchq)], didx)

            @pl.loop(0, nchq)
            def _(c):
                pltpu.sync_copy(ones, acc.at[didx.at[c]], add=True)

        plsc.subcore_barrier()
        _acc_to_out(acc, out_hbm, cid, sid, rpt)

    k(dst_rs, out_ref)


def _sc_agg(y, src_rs, dst_rs, out_ref, accn):
    nch = src_rs.shape[1]
    nch_h = nch // _NC
    nchq = nch_h // 2
    rpt = accn // _NS

    @functools.partial(
        pl.kernel,
        out_type=(),
        mesh=_mesh(),
        scratch_types=[
            pltpu.VMEM((nchq, _CH), jnp.int32),
            pltpu.VMEM((nchq, _CH), jnp.int32),
            pltpu.VMEM((32, 128), _F32),
            pltpu.VMEM_SHARED((accn, 128), _F32),
        ] + [pltpu.VMEM((_CH, 128), _F32)] * _NBUF
          + [pltpu.SemaphoreType.DMA] * _NBUF,
    )
    def k(y_hbm, src_hbm, dst_hbm, out_hbm, sidx, didx, zbuf, acc, *bufsem):
        bufs = bufsem[:_NBUF]
        sg = bufsem[_NBUF:2 * _NBUF]
        cid = lax.axis_index("c")
        sid = lax.axis_index("s")

        _fill_vmem_f32(zbuf, 32, 0.0)
        _zero_acc(acc, zbuf, sid, rpt)
        plsc.subcore_barrier()

        for h in range(2):
            off = cid * nch_h + h * nchq
            pltpu.sync_copy(src_hbm.at[sid, pl.ds(off, nchq)], sidx)
            pltpu.sync_copy(dst_hbm.at[sid, pl.ds(off, nchq)], didx)

            for j in range(_NBUF):
                pltpu.async_copy(y_hbm.at[sidx.at[j]], bufs[j], sg[j])

            @pl.loop(0, nchq // _NBUF)
            def _(i):
                c0 = _NBUF * i
                for j in range(_NBUF):
                    c = c0 + j
                    pltpu.make_async_copy(y_hbm.at[sidx.at[c]],
                                          bufs[j], sg[j]).wait()
                    pltpu.sync_copy(bufs[j], acc.at[didx.at[c]], add=True)

                    @pl.when(c + _NBUF < nchq)
                    def _():
                        pltpu.async_copy(y_hbm.at[sidx.at[c + _NBUF]],
                                         bufs[j], sg[j])

        plsc.subcore_barrier()
        _acc_to_out(acc, out_hbm, cid, sid, rpt)

    k(y, src_rs, dst_rs, out_ref)


def _dinv_block(degp_ref):
    d = degp_ref[0, :, 0:1] + degp_ref[1, :, 0:1] + 1.0
    return lax.rsqrt(d)


def _tc_p1(x, W1, degp, bn):
    n = x.shape[0]

    def body(x_ref, w_ref, degp_ref, y_ref):
        dinv = _dinv_block(degp_ref)
        y_ref[...] = dinv * jnp.dot(x_ref[...], w_ref[...],
                                    preferred_element_type=_F32)

    return pl.pallas_call(
        body,
        grid=(n // bn,),
        in_specs=[
            pl.BlockSpec((bn, 128), lambda i: (i, 0)),
            pl.BlockSpec((128, 128), lambda i: (0, 0)),
            pl.BlockSpec((2, bn, 128), lambda i: (0, i, 0)),
        ],
        out_specs=pl.BlockSpec((bn, 128), lambda i: (i, 0)),
        out_shape=jax.ShapeDtypeStruct((n, 128), _F32),
    )(x, W1, degp)


def _tc_p3(agg1, y1, degp, W2, b1, bn):
    n = y1.shape[0]

    def body(agg_ref, y_ref, degp_ref, w_ref, b_ref, y2_ref):
        dinv = _dinv_block(degp_ref)
        s = agg_ref[0] + agg_ref[1] + y_ref[...]
        z = jnp.maximum(dinv * s + b_ref[...], 0.0)
        y2_ref[...] = dinv * jnp.dot(z, w_ref[...], preferred_element_type=_F32)

    return pl.pallas_call(
        body,
        grid=(n // bn,),
        in_specs=[
            pl.BlockSpec((2, bn, 128), lambda i: (0, i, 0)),
            pl.BlockSpec((bn, 128), lambda i: (i, 0)),
            pl.BlockSpec((2, bn, 128), lambda i: (0, i, 0)),
            pl.BlockSpec((128, 128), lambda i: (0, 0)),
            pl.BlockSpec((1, 128), lambda i: (0, 0)),
        ],
        out_specs=pl.BlockSpec((bn, 128), lambda i: (i, 0)),
        out_shape=jax.ShapeDtypeStruct((n, 128), _F32),
    )(agg1, y1, degp, W2, b1)


def _tc_p5(agg2, y2, degp, b2, bn):
    n = y2.shape[0]

    def body(agg_ref, y_ref, degp_ref, b_ref, o_ref):
        dinv = _dinv_block(degp_ref)
        s = agg_ref[0] + agg_ref[1] + y_ref[...]
        o_ref[...] = dinv * s + b_ref[...]

    return pl.pallas_call(
        body,
        grid=(n // bn,),
        in_specs=[
            pl.BlockSpec((2, bn, 128), lambda i: (0, i, 0)),
            pl.BlockSpec((bn, 128), lambda i: (i, 0)),
            pl.BlockSpec((2, bn, 128), lambda i: (0, i, 0)),
            pl.BlockSpec((1, 128), lambda i: (0, 0)),
        ],
        out_specs=pl.BlockSpec((bn, 128), lambda i: (i, 0)),
        out_shape=jax.ShapeDtypeStruct((n, 128), _F32),
    )(agg2, y2, degp, b2)


def kernel(x, edge_index, W1, b1, W2, b2):
    n = x.shape[0]
    e = edge_index.shape[1]
    src = edge_index[0]
    dst = edge_index[1]
    accn = n + _PADR
    assert accn % (_NS * 64) == 0 and n % 1000 == 0

    nch = -(-e // (_NS * _CH))
    nch = -(-nch // (2 * _NBUF)) * (2 * _NBUF)
    npad = _NS * nch * _CH - e

    pad_src = (jnp.arange(npad, dtype=jnp.int32) * 37) % n
    pad_dst = n + (jnp.arange(npad, dtype=jnp.int32) % _PADR)
    src_rs = jnp.concatenate([src, pad_src]).reshape(_NS, nch, _CH)
    dst_rs = jnp.concatenate([dst, pad_dst]).reshape(_NS, nch, _CH)

    bn = 1000
    b1r = b1.reshape(1, 128)
    b2r = b2.reshape(1, 128)

    degp_ref = jax.new_ref(jnp.zeros((_NC, accn, 128), _F32))
    _sc_deg(dst_rs, degp_ref, accn)
    degp = degp_ref[...]

    y1 = _tc_p1(x, W1, degp, bn)

    agg1_ref = jax.new_ref(jnp.zeros((_NC, accn, 128), _F32))
    _sc_agg(y1, src_rs, dst_rs, agg1_ref, accn)
    y2 = _tc_p3(agg1_ref[...], y1, degp, W2, b1r, bn)

    agg2_ref = jax.new_ref(jnp.zeros((_NC, accn, 128), _F32))
    _sc_agg(y2, src_rs, dst_rs, agg2_ref, accn)
    out = _tc_p5(agg2_ref[...], y2, degp, b2r, bn)
    return out
